# baseline (speedup 1.0000x reference)
import functools

import jax
import jax.numpy as jnp
from jax import lax
from jax.experimental import pallas as pl
from jax.experimental.pallas import tpu as pltpu
from jax.experimental.pallas import tpu_sc as plsc

_NC, _NS = 2, 16
_NBUF = 4


def _emb_kernel(bsz, l, d, scale):
    nw = _NC * _NS
    nb = bsz // nw
    ngrp = nb // _NBUF

    mesh = plsc.VectorSubcoreMesh(
        core_axis_name="c", subcore_axis_name="s",
        num_cores=_NC, num_subcores=_NS,
    )

    @functools.partial(
        pl.kernel,
        out_type=jax.ShapeDtypeStruct((bsz, l, d), jnp.float32),
        mesh=mesh,
        scratch_types=[
            pltpu.VMEM((nb, l), jnp.int32),
            pltpu.VMEM((_NBUF, l, d), jnp.float32),
            pltpu.SemaphoreType.DMA((_NBUF,)),
            pltpu.SemaphoreType.DMA((_NBUF,)),
        ],
        compiler_params=pltpu.CompilerParams(use_tc_tiling_on_sc=True),
    )
    def emb(idx_hbm, table_hbm, out_hbm, idx_v, rows_v, gsem, osem):
        wid = lax.axis_index("s") * _NC + lax.axis_index("c")
        base = wid * nb
        pltpu.sync_copy(idx_hbm.at[wid], idx_v)

        def group(g, carry):
            c0 = g * _NBUF
            for b in range(_NBUF):
                @pl.when(g > 0)
                def _wait_out(b=b):
                    pltpu.make_async_copy(
                        rows_v.at[b], out_hbm.at[base + c0 + b], osem.at[b]
                    ).wait()
                pltpu.async_copy(
                    table_hbm.at[idx_v.at[c0 + b]], rows_v.at[b], gsem.at[b]
                )
            for b in range(_NBUF):
                pltpu.make_async_copy(
                    table_hbm.at[idx_v.at[c0 + b]], rows_v.at[b], gsem.at[b]
                ).wait()

                def srow(r, carry2, b=b):
                    for j in range(d // 16):
                        sl = pl.ds(j * 16, 16)
                        rows_v[b, r, sl] = rows_v[b, r, sl] * scale
                    return carry2

                lax.fori_loop(0, l, srow, 0, unroll=False)
                pltpu.async_copy(
                    rows_v.at[b], out_hbm.at[base + c0 + b], osem.at[b]
                )
            return carry

        lax.fori_loop(0, ngrp, group, 0, unroll=False)
        for b in range(_NBUF):
            pltpu.make_async_copy(
                rows_v.at[b], out_hbm.at[base + b], osem.at[b]
            ).wait()

    return emb


def kernel(x, embedding_table):
    bsz, l = x.shape
    v, d = embedding_table.shape
    nw = _NC * _NS
    scale = float(d) ** 0.5

    idx = x.reshape(nw, bsz // nw, l).astype(jnp.int32)
    return _emb_kernel(bsz, l, d, scale)(idx, embedding_table)


# speedup vs baseline: 1.8751x; 1.8751x over previous
import functools

import jax
import jax.numpy as jnp
from jax import lax
from jax.experimental import pallas as pl
from jax.experimental.pallas import tpu as pltpu
from jax.experimental.pallas import tpu_sc as plsc

_NC, _NS = 2, 16
_SPLIT = 2
_NBUF = 10


def _emb_kernel(bsz, l, d, scale):
    nw = _NC * _NS
    nb = bsz // nw
    ch = nb // _SPLIT
    lg = _NBUF // _SPLIT
    ngrp = l // lg

    mesh = plsc.VectorSubcoreMesh(
        core_axis_name="c", subcore_axis_name="s",
        num_cores=_NC, num_subcores=_NS,
    )

    @functools.partial(
        pl.kernel,
        out_type=jax.ShapeDtypeStruct((l, bsz, d), jnp.float32),
        mesh=mesh,
        scratch_types=[
            pltpu.VMEM((l, nb), jnp.int32),
            pltpu.VMEM((_NBUF, ch, d), jnp.float32),
            pltpu.SemaphoreType.DMA((_NBUF,)),
            pltpu.SemaphoreType.DMA((_NBUF,)),
        ],
    )
    def emb(idx_hbm, table_hbm, out_hbm, idx_v, rows_v, gsem, osem):
        wid = lax.axis_index("s") * _NC + lax.axis_index("c")
        base = wid * nb
        pltpu.sync_copy(idx_hbm.at[wid], idx_v)

        def chunk_refs(g, b):
            li = g * lg + b // _SPLIT
            h = b % _SPLIT
            idx_sl = idx_v.at[li, pl.ds(h * ch, ch)]
            out_sl = out_hbm.at[li, pl.ds(base + h * ch, ch)]
            return idx_sl, out_sl

        def group(g, carry):
            for b in range(_NBUF):
                idx_sl, out_sl = chunk_refs(g, b)

                @pl.when(g > 0)
                def _wait_out(b=b, out_sl=out_sl):
                    pltpu.make_async_copy(
                        rows_v.at[b], out_sl, osem.at[b]
                    ).wait()

                pltpu.async_copy(table_hbm.at[idx_sl], rows_v.at[b], gsem.at[b])
            for b in range(_NBUF):
                idx_sl, out_sl = chunk_refs(g, b)
                pltpu.make_async_copy(
                    table_hbm.at[idx_sl], rows_v.at[b], gsem.at[b]
                ).wait()

                def srow(r, carry2, b=b):
                    for j in range(d // 16):
                        sl = pl.ds(j * 16, 16)
                        rows_v[b, r, sl] = rows_v[b, r, sl] * scale
                    return carry2

                lax.fori_loop(0, ch, srow, 0, unroll=False)
                pltpu.async_copy(rows_v.at[b], out_sl, osem.at[b])
            return carry

        lax.fori_loop(0, ngrp, group, 0, unroll=False)
        for b in range(_NBUF):
            _, out_sl = chunk_refs(ngrp - 1, b)
            pltpu.make_async_copy(rows_v.at[b], out_sl, osem.at[b]).wait()

    return emb


def kernel(x, embedding_table):
    bsz, l = x.shape
    v, d = embedding_table.shape
    nw = _NC * _NS
    nb = bsz // nw
    scale = float(d) ** 0.5

    idx = x.reshape(nw, nb, l).transpose(0, 2, 1).astype(jnp.int32)
    out = _emb_kernel(bsz, l, d, scale)(idx, embedding_table)
    return out.transpose(1, 0, 2)
